# Initial kernel scaffold; baseline (speedup 1.0000x reference)
import functools

import jax
import jax.numpy as jnp
from jax import lax
from jax.experimental import pallas as pl
from jax.experimental.pallas import tpu as pltpu
from jax.experimental.pallas import tpu_sc as plsc

IN_F = 16384
OUT_F = 16384
K = 64

NC = 2
NS = 16
NW = NC * NS
CHUNK = 128
ROWS_PER_SUB = IN_F // NS


def _make_sc_spmm(num_chunks):
    mesh = plsc.VectorSubcoreMesh(core_axis_name="c", subcore_axis_name="s")

    @functools.partial(
        pl.kernel,
        out_type=jax.ShapeDtypeStruct((NC, IN_F, K), jnp.float32),
        mesh=mesh,
        scratch_types=[
            pltpu.VMEM((num_chunks, CHUNK), jnp.int32),
            pltpu.VMEM((num_chunks, CHUNK), jnp.int32),
            pltpu.VMEM((num_chunks, CHUNK), jnp.float32),
            pltpu.VMEM((CHUNK, K), jnp.float32),
            pltpu.VMEM_SHARED((IN_F, K), jnp.float32),
            pltpu.SemaphoreType.DMA,
        ],
    )
    def sc_spmm(x_hbm, rows_hbm, cols_hbm, vals_hbm, out_hbm,
                rows_v, cols_v, vals_v, gath, acc, sem):
        c = lax.axis_index("c")
        s = lax.axis_index("s")
        wid = c * NS + s

        pltpu.sync_copy(rows_hbm.at[wid], rows_v)
        pltpu.sync_copy(cols_hbm.at[wid], cols_v)
        pltpu.sync_copy(vals_hbm.at[wid], vals_v)

        zero = jnp.zeros((16,), jnp.float32)

        def zbody(n, carry):
            for j in range(K // 16):
                gath[n, pl.ds(j * 16, 16)] = zero
            return carry

        lax.fori_loop(0, CHUNK, zbody, 0)
        for t in range(ROWS_PER_SUB // CHUNK):
            pltpu.sync_copy(gath, acc.at[pl.ds(s * ROWS_PER_SUB + t * CHUNK, CHUNK)])
        plsc.subcore_barrier()

        def chunk_body(k, carry):
            pltpu.async_copy(x_hbm.at[cols_v.at[k]], gath, sem).wait()

            def nbody(n, cc):
                v = vals_v[k, n]
                for j in range(K // 16):
                    sl = pl.ds(j * 16, 16)
                    gath[n, sl] = gath[n, sl] * v
                return cc

            lax.fori_loop(0, CHUNK, nbody, 0)

            pltpu.sync_copy(gath, acc.at[rows_v.at[k]], add=True)
            return carry

        lax.fori_loop(0, num_chunks, chunk_body, 0)
        plsc.subcore_barrier()

        for t in range(ROWS_PER_SUB // CHUNK):
            off = s * ROWS_PER_SUB + t * CHUNK
            pltpu.sync_copy(acc.at[pl.ds(off, CHUNK)],
                            out_hbm.at[c, pl.ds(off, CHUNK)])

    return sc_spmm


def _combine_body(p_ref, b_ref, o_ref):
    o_ref[...] = p_ref[0] + p_ref[1] + b_ref[...]


@jax.jit
def kernel(x, rows, cols, vals, bias):
    nnz = rows.shape[0]
    num_chunks = -(-nnz // (NW * CHUNK))
    padded = NW * num_chunks * CHUNK
    pad = padded - nnz

    rows_p = jnp.pad(rows.astype(jnp.int32), (0, pad)).reshape(NW, num_chunks, CHUNK)
    cols_p = jnp.pad(cols.astype(jnp.int32), (0, pad)).reshape(NW, num_chunks, CHUNK)
    vals_p = jnp.pad(vals, (0, pad)).reshape(NW, num_chunks, CHUNK)

    partial = _make_sc_spmm(num_chunks)(x, rows_p, cols_p, vals_p)

    out = pl.pallas_call(
        _combine_body,
        out_shape=jax.ShapeDtypeStruct((IN_F, K), jnp.float32),
        grid=(IN_F // 1024,),
        in_specs=[
            pl.BlockSpec((NC, 1024, K), lambda i: (0, i, 0)),
            pl.BlockSpec((1024, 1), lambda i: (i, 0)),
        ],
        out_specs=pl.BlockSpec((1024, K), lambda i: (i, 0)),
    )(partial, bias.reshape(IN_F, 1))
    return out


# speedup vs baseline: 5.9927x; 5.9927x over previous
import functools

import jax
import jax.numpy as jnp
from jax import lax
from jax.experimental import pallas as pl
from jax.experimental.pallas import tpu as pltpu
from jax.experimental.pallas import tpu_sc as plsc

IN_F = 16384
OUT_F = 16384
K = 64

NC = 2
NS = 16
NW = NC * NS
CHUNK = 128
ROWS_PER_SUB = IN_F // NS


def _make_sc_spmm(num_chunks):
    mesh = plsc.VectorSubcoreMesh(core_axis_name="c", subcore_axis_name="s")

    @functools.partial(
        pl.kernel,
        out_type=jax.ShapeDtypeStruct((NC, IN_F, K), jnp.float32),
        mesh=mesh,
        scratch_types=[
            pltpu.VMEM((num_chunks, CHUNK), jnp.int32),
            pltpu.VMEM((num_chunks, CHUNK), jnp.int32),
            pltpu.VMEM((num_chunks, CHUNK), jnp.float32),
            pltpu.VMEM((CHUNK, K), jnp.float32),
            pltpu.VMEM_SHARED((IN_F, K), jnp.float32),
            pltpu.SemaphoreType.DMA,
        ],
        compiler_params=pltpu.CompilerParams(use_tc_tiling_on_sc=False),
    )
    def sc_spmm(x_hbm, rows_hbm, cols_hbm, vals_hbm, out_hbm,
                rows_v, cols_v, vals_v, gath, acc, sem):
        c = lax.axis_index("c")
        s = lax.axis_index("s")
        wid = c * NS + s

        pltpu.sync_copy(rows_hbm.at[wid], rows_v)
        pltpu.sync_copy(cols_hbm.at[wid], cols_v)
        pltpu.sync_copy(vals_hbm.at[wid], vals_v)

        zero = jnp.zeros((16,), jnp.float32)

        def zbody(n, carry):
            for j in range(K // 16):
                gath[n, pl.ds(j * 16, 16)] = zero
            return carry

        lax.fori_loop(0, CHUNK, zbody, 0)
        for t in range(ROWS_PER_SUB // CHUNK):
            pltpu.sync_copy(gath, acc.at[pl.ds(s * ROWS_PER_SUB + t * CHUNK, CHUNK)])
        plsc.subcore_barrier()

        def chunk_body(k, carry):
            pltpu.async_copy(x_hbm.at[cols_v.at[k]], gath, sem).wait()

            def gbody(g, cc):
                vvec = vals_v[k, pl.ds(g * 16, 16)]
                for i in range(16):
                    v = vvec[i]
                    n = g * 16 + i
                    for j in range(K // 16):
                        sl = pl.ds(j * 16, 16)
                        gath[n, sl] = gath[n, sl] * v
                return cc

            lax.fori_loop(0, CHUNK // 16, gbody, 0)

            pltpu.sync_copy(gath, acc.at[rows_v.at[k]], add=True)
            return carry

        lax.fori_loop(0, num_chunks, chunk_body, 0)
        plsc.subcore_barrier()

        for t in range(ROWS_PER_SUB // CHUNK):
            off = s * ROWS_PER_SUB + t * CHUNK
            pltpu.sync_copy(acc.at[pl.ds(off, CHUNK)],
                            out_hbm.at[c, pl.ds(off, CHUNK)])

    return sc_spmm


def _combine_body(p_ref, b_ref, o_ref):
    o_ref[...] = p_ref[0] + p_ref[1] + b_ref[...]


@jax.jit
def kernel(x, rows, cols, vals, bias):
    nnz = rows.shape[0]
    num_chunks = -(-nnz // (NW * CHUNK))
    padded = NW * num_chunks * CHUNK
    pad = padded - nnz

    rows_p = jnp.pad(rows.astype(jnp.int32), (0, pad)).reshape(NW, num_chunks, CHUNK)
    cols_p = jnp.pad(cols.astype(jnp.int32), (0, pad)).reshape(NW, num_chunks, CHUNK)
    vals_p = jnp.pad(vals, (0, pad)).reshape(NW, num_chunks, CHUNK)

    partial = _make_sc_spmm(num_chunks)(x, rows_p, cols_p, vals_p)

    out = pl.pallas_call(
        _combine_body,
        out_shape=jax.ShapeDtypeStruct((IN_F, K), jnp.float32),
        grid=(IN_F // 1024,),
        in_specs=[
            pl.BlockSpec((NC, 1024, K), lambda i: (0, i, 0)),
            pl.BlockSpec((1024, 1), lambda i: (i, 0)),
        ],
        out_specs=pl.BlockSpec((1024, K), lambda i: (i, 0)),
    )(partial, bias.reshape(IN_F, 1))
    return out
